# Initial kernel scaffold; baseline (speedup 1.0000x reference)
import jax
import jax.numpy as jnp
from jax.experimental import pallas as pl


def kernel(inputs, relative_biases):
    raise NotImplementedError("write your pallas kernel here")


# speedup vs baseline: 254.9435x; 254.9435x over previous
import jax
import jax.numpy as jnp
from jax.experimental import pallas as pl
from jax.experimental.pallas import tpu as pltpu

_TQ = 256
_TK = 256


def _add_bias_body(f_ref, x_ref, o_ref):
    f = f_ref[0, 0, :]
    fb = jnp.broadcast_to(f[None, :], (_TQ, 512))
    bias = pltpu.roll(fb, 257, axis=1, stride=1, stride_axis=0)[:, :_TK]
    o_ref[...] = x_ref[...] + bias[None, :, :]


def kernel(inputs, relative_biases):
    t = relative_biases
    e = jnp.concatenate(
        [jnp.full((1919,), t[0], t.dtype), t, jnp.full((1920,), t[256], t.dtype)]
    )
    frames = e.reshape(16, 256)
    f_all = jnp.concatenate([frames[:15], frames[1:]], axis=1)
    f_all = f_all.reshape(15, 1, 512)

    b, s_q, s_k = inputs.shape
    grid = (s_q // _TQ, s_k // _TK)
    return pl.pallas_call(
        _add_bias_body,
        grid=grid,
        in_specs=[
            pl.BlockSpec((1, 1, 512), lambda qi, ki: (ki - qi + 7, 0, 0)),
            pl.BlockSpec((b, _TQ, _TK), lambda qi, ki: (0, qi, ki)),
        ],
        out_specs=pl.BlockSpec((b, _TQ, _TK), lambda qi, ki: (0, qi, ki)),
        out_shape=jax.ShapeDtypeStruct(inputs.shape, inputs.dtype),
    )(f_all, inputs)
